# Initial kernel scaffold; baseline (speedup 1.0000x reference)
import jax
import jax.numpy as jnp
from jax.experimental import pallas as pl


def kernel(words, chars, word_table, trainable_table, char_table, conv_w, conv_b, hw1_wt, hw1_bt, hw1_wh, hw1_bh, hw2_wt, hw2_bt, hw2_wh, hw2_bh):
    raise NotImplementedError("write your pallas kernel here")


# speedup vs baseline: 1.5278x; 1.5278x over previous
import functools

import jax
import jax.numpy as jnp
from jax import lax
from jax.experimental import pallas as pl
from jax.experimental.pallas import tpu as pltpu
from jax.experimental.pallas import tpu_sc as plsc

V = 100001
T_TBL = 1001
CV = 257
WD = 300
CD = 200
K = 5
HD = WD + CD
WORD_RANGE = V - T_TBL
B, L, C = 64, 200, 16
NTOK = B * L

_NC, _NS = 2, 16
_NW = _NC * _NS
_BPW = NTOK // _NW
_CHUNK = 80
_NCH = _BPW // _CHUNK


WDP = 384


def _sc_gather_body(table_hbm, idx_hbm, out_hbm, idx_v, rows_v, sem):
    wid = lax.axis_index("s") * _NC + lax.axis_index("c")
    base = wid * _BPW
    for j in range(_NCH):
        pltpu.sync_copy(idx_hbm.at[pl.ds(base + j * _CHUNK, _CHUNK)], idx_v.at[j])
    for j in range(_NCH):
        pltpu.async_copy(
            table_hbm.at[idx_v.at[j]],
            rows_v.at[j % 2],
            sem,
        ).wait()
        pltpu.sync_copy(rows_v.at[j % 2], out_hbm.at[pl.ds(base + j * _CHUNK, _CHUNK)])


def _sc_word_gather(word_table_padded, flat_words):
    mesh = plsc.VectorSubcoreMesh(core_axis_name="c", subcore_axis_name="s")
    f = pl.kernel(
        _sc_gather_body,
        out_type=jax.ShapeDtypeStruct((NTOK, WDP), jnp.float32),
        mesh=mesh,
        scratch_types=[
            pltpu.VMEM((_NCH, _CHUNK), jnp.int32),
            pltpu.VMEM((2, _CHUNK, WDP), jnp.float32),
            pltpu.SemaphoreType.DMA,
        ],
    )
    return f(word_table_padded, flat_words)


TBLK = 256
GRID = NTOK // TBLK


def _tc_body(we_ref, w_ref, ch_ref, tt_ref, ct_ref, cw_ref, cb_ref,
             w1t_ref, b1t_ref, w1h_ref, b1h_ref,
             w2t_ref, b2t_ref, w2h_ref, b2h_ref,
             out_ref, cep_ref):
    f32 = jnp.float32
    words = w_ref[0, 0, :]
    tidx = jnp.maximum(words - WORD_RANGE, 0)
    oh_t = (lax.broadcasted_iota(jnp.int32, (TBLK, T_TBL), 1)
            == tidx[:, None]).astype(f32)
    te = jnp.dot(oh_t, tt_ref[...], preferred_element_type=f32)
    x_we = we_ref[...][:, :WD] + jnp.maximum(te, 0.0)

    ch = ch_ref[0]
    oh_c = (lax.broadcasted_iota(jnp.int32, (TBLK, C, CV), 2)
            == ch[:, :, None]).astype(f32)
    ce = jnp.dot(oh_c.reshape(TBLK * C, CV), ct_ref[...],
                 preferred_element_type=f32)

    cep_ref[:, 0:2, :] = jnp.zeros((TBLK, 2, CD), f32)
    cep_ref[:, 2:2 + C, :] = ce.reshape(TBLK, C, CD)
    cep_ref[:, 2 + C:, :] = jnp.zeros((TBLK, 2, CD), f32)

    acc = jnp.zeros((TBLK * C, CD), f32)
    for k in range(K):
        sl = cep_ref[:, k:k + C, :].reshape(TBLK * C, CD)
        acc = acc + jnp.dot(sl, cw_ref[k], preferred_element_type=f32)
    acc = jnp.maximum(acc + cb_ref[0], 0.0)
    cpool = jnp.max(acc.reshape(TBLK, C, CD), axis=1)

    x = jnp.concatenate([x_we, cpool], axis=1)
    for wt_ref, bt_ref, wh_ref, bh_ref in (
            (w1t_ref, b1t_ref, w1h_ref, b1h_ref),
            (w2t_ref, b2t_ref, w2h_ref, b2h_ref)):
        t = jax.nn.sigmoid(jnp.dot(x, wt_ref[...], preferred_element_type=f32)
                           + bt_ref[0])
        h = jnp.maximum(jnp.dot(x, wh_ref[...], preferred_element_type=f32)
                        + bh_ref[0], 0.0)
        x = h * t + x * (1.0 - t)
    out_ref[...] = x


def _full(shape):
    nd = len(shape)
    return pl.BlockSpec(shape, lambda i, _n=nd: (0,) * _n)


def _tc_fused(we, flat_words, chars2d, trainable_table, char_table, conv_w,
              conv_b, hw1_wt, hw1_bt, hw1_wh, hw1_bh, hw2_wt, hw2_bt,
              hw2_wh, hw2_bh, interpret=False):
    words_r = flat_words.reshape(GRID, 1, TBLK)
    chars_r = chars2d.reshape(GRID, TBLK, C)
    return pl.pallas_call(
        _tc_body,
        grid=(GRID,),
        in_specs=[
            pl.BlockSpec((TBLK, WDP), lambda i: (i, 0)),
            pl.BlockSpec((1, 1, TBLK), lambda i: (i, 0, 0)),
            pl.BlockSpec((1, TBLK, C), lambda i: (i, 0, 0)),
            _full((T_TBL, WD)),
            _full((CV, CD)),
            _full((K, CD, CD)),
            _full((1, CD)),
            _full((HD, HD)), _full((1, HD)), _full((HD, HD)), _full((1, HD)),
            _full((HD, HD)), _full((1, HD)), _full((HD, HD)), _full((1, HD)),
        ],
        out_specs=pl.BlockSpec((TBLK, HD), lambda i: (i, 0)),
        out_shape=jax.ShapeDtypeStruct((NTOK, HD), jnp.float32),
        scratch_shapes=[pltpu.VMEM((TBLK, C + 4, CD), jnp.float32)],
        interpret=interpret,
    )(we, words_r, chars_r, trainable_table, char_table, conv_w,
      conv_b.reshape(1, CD),
      hw1_wt, hw1_bt.reshape(1, HD), hw1_wh, hw1_bh.reshape(1, HD),
      hw2_wt, hw2_bt.reshape(1, HD), hw2_wh, hw2_bh.reshape(1, HD))


def kernel(words, chars, word_table, trainable_table, char_table, conv_w,
           conv_b, hw1_wt, hw1_bt, hw1_wh, hw1_bh, hw2_wt, hw2_bt, hw2_wh,
           hw2_bh):
    flat_words = words.reshape(NTOK)
    chars2d = chars.reshape(NTOK, C)
    wt_pad = jnp.pad(word_table, ((0, 0), (0, WDP - WD)))
    we = _sc_word_gather(wt_pad, flat_words)
    out = _tc_fused(we, flat_words, chars2d, trainable_table, char_table,
                    conv_w, conv_b, hw1_wt, hw1_bt, hw1_wh, hw1_bh,
                    hw2_wt, hw2_bt, hw2_wh, hw2_bh)
    return out.reshape(B, L, HD)


# speedup vs baseline: 2.2914x; 1.4998x over previous
import functools

import jax
import jax.numpy as jnp
from jax import lax
from jax.experimental import pallas as pl
from jax.experimental.pallas import tpu as pltpu
from jax.experimental.pallas import tpu_sc as plsc

V = 100001
T_TBL = 1001
CV = 257
WD = 300
CD = 200
K = 5
HD = WD + CD
WORD_RANGE = V - T_TBL
B, L, C = 64, 200, 16
NTOK = B * L

_NC, _NS = 2, 16
_NW = _NC * _NS
_BPW = NTOK // _NW
_CHUNK = 80
_NCH = _BPW // _CHUNK


WDP = 384

_PADBLK = 1024
_PADGRID = -(-V // _PADBLK)


def _pad_body(src_ref, dst_ref):
    dst_ref[:, :WD] = src_ref[...]
    dst_ref[:, WD:] = jnp.zeros((_PADBLK, WDP - WD), jnp.float32)


def _pad_table(word_table):
    return pl.pallas_call(
        _pad_body,
        grid=(_PADGRID,),
        in_specs=[pl.BlockSpec((_PADBLK, WD), lambda i: (i, 0))],
        out_specs=pl.BlockSpec((_PADBLK, WDP), lambda i: (i, 0)),
        out_shape=jax.ShapeDtypeStruct((V, WDP), jnp.float32),
    )(word_table)


def _sc_gather_body(table_hbm, idx_hbm, out_hbm, idx_v, rows_v, sem):
    wid = lax.axis_index("s") * _NC + lax.axis_index("c")
    base = wid * _BPW
    for j in range(_NCH):
        pltpu.sync_copy(idx_hbm.at[pl.ds(base + j * _CHUNK, _CHUNK)], idx_v.at[j])
    for j in range(_NCH):
        pltpu.async_copy(
            table_hbm.at[idx_v.at[j]],
            rows_v.at[j % 2],
            sem,
        ).wait()
        pltpu.sync_copy(rows_v.at[j % 2], out_hbm.at[pl.ds(base + j * _CHUNK, _CHUNK)])


def _sc_word_gather(word_table_padded, flat_words):
    mesh = plsc.VectorSubcoreMesh(core_axis_name="c", subcore_axis_name="s")
    f = pl.kernel(
        _sc_gather_body,
        out_type=jax.ShapeDtypeStruct((NTOK, WDP), jnp.float32),
        mesh=mesh,
        scratch_types=[
            pltpu.VMEM((_NCH, _CHUNK), jnp.int32),
            pltpu.VMEM((2, _CHUNK, WDP), jnp.float32),
            pltpu.SemaphoreType.DMA,
        ],
    )
    return f(word_table_padded, flat_words)


TBLK = 256
GRID = NTOK // TBLK


def _tc_body(we_ref, w_ref, ch_ref, tt_ref, ct_ref, cw_ref, cb_ref,
             w1t_ref, b1t_ref, w1h_ref, b1h_ref,
             w2t_ref, b2t_ref, w2h_ref, b2h_ref,
             out_ref, cep_ref):
    f32 = jnp.float32
    words = w_ref[0, 0, :]
    tidx = jnp.maximum(words - WORD_RANGE, 0)
    oh_t = (lax.broadcasted_iota(jnp.int32, (TBLK, T_TBL), 1)
            == tidx[:, None]).astype(f32)
    te = jnp.dot(oh_t, tt_ref[...], preferred_element_type=f32)
    x_we = we_ref[...][:, :WD] + jnp.maximum(te, 0.0)

    ch = ch_ref[0]
    oh_c = (lax.broadcasted_iota(jnp.int32, (TBLK, C, CV), 2)
            == ch[:, :, None]).astype(f32)
    ce = jnp.dot(oh_c.reshape(TBLK * C, CV), ct_ref[...],
                 preferred_element_type=f32)

    cep_ref[:, 0:2, :] = jnp.zeros((TBLK, 2, CD), f32)
    cep_ref[:, 2:2 + C, :] = ce.reshape(TBLK, C, CD)
    cep_ref[:, 2 + C:, :] = jnp.zeros((TBLK, 2, CD), f32)

    acc = jnp.zeros((TBLK * C, CD), f32)
    for k in range(K):
        sl = cep_ref[:, k:k + C, :].reshape(TBLK * C, CD)
        acc = acc + jnp.dot(sl, cw_ref[k], preferred_element_type=f32)
    acc = jnp.maximum(acc + cb_ref[0], 0.0)
    cpool = jnp.max(acc.reshape(TBLK, C, CD), axis=1)

    x = jnp.concatenate([x_we, cpool], axis=1)
    for wt_ref, bt_ref, wh_ref, bh_ref in (
            (w1t_ref, b1t_ref, w1h_ref, b1h_ref),
            (w2t_ref, b2t_ref, w2h_ref, b2h_ref)):
        t = jax.nn.sigmoid(jnp.dot(x, wt_ref[...], preferred_element_type=f32)
                           + bt_ref[0])
        h = jnp.maximum(jnp.dot(x, wh_ref[...], preferred_element_type=f32)
                        + bh_ref[0], 0.0)
        x = h * t + x * (1.0 - t)
    out_ref[...] = x


def _full(shape):
    nd = len(shape)
    return pl.BlockSpec(shape, lambda i, _n=nd: (0,) * _n)


def _tc_fused(we, flat_words, chars2d, trainable_table, char_table, conv_w,
              conv_b, hw1_wt, hw1_bt, hw1_wh, hw1_bh, hw2_wt, hw2_bt,
              hw2_wh, hw2_bh, interpret=False):
    words_r = flat_words.reshape(GRID, 1, TBLK)
    chars_r = chars2d.reshape(GRID, TBLK, C)
    return pl.pallas_call(
        _tc_body,
        grid=(GRID,),
        in_specs=[
            pl.BlockSpec((TBLK, WDP), lambda i: (i, 0)),
            pl.BlockSpec((1, 1, TBLK), lambda i: (i, 0, 0)),
            pl.BlockSpec((1, TBLK, C), lambda i: (i, 0, 0)),
            _full((T_TBL, WD)),
            _full((CV, CD)),
            _full((K, CD, CD)),
            _full((1, CD)),
            _full((HD, HD)), _full((1, HD)), _full((HD, HD)), _full((1, HD)),
            _full((HD, HD)), _full((1, HD)), _full((HD, HD)), _full((1, HD)),
        ],
        out_specs=pl.BlockSpec((TBLK, HD), lambda i: (i, 0)),
        out_shape=jax.ShapeDtypeStruct((NTOK, HD), jnp.float32),
        scratch_shapes=[pltpu.VMEM((TBLK, C + 4, CD), jnp.float32)],
        interpret=interpret,
    )(we, words_r, chars_r, trainable_table, char_table, conv_w,
      conv_b.reshape(1, CD),
      hw1_wt, hw1_bt.reshape(1, HD), hw1_wh, hw1_bh.reshape(1, HD),
      hw2_wt, hw2_bt.reshape(1, HD), hw2_wh, hw2_bh.reshape(1, HD))


def kernel(words, chars, word_table, trainable_table, char_table, conv_w,
           conv_b, hw1_wt, hw1_bt, hw1_wh, hw1_bh, hw2_wt, hw2_bt, hw2_wh,
           hw2_bh):
    flat_words = words.reshape(NTOK)
    chars2d = chars.reshape(NTOK, C)
    wt_pad = _pad_table(word_table)
    we = _sc_word_gather(wt_pad, flat_words)
    out = _tc_fused(we, flat_words, chars2d, trainable_table, char_table,
                    conv_w, conv_b, hw1_wt, hw1_bt, hw1_wh, hw1_bh,
                    hw2_wt, hw2_bt, hw2_wh, hw2_bh)
    return out.reshape(B, L, HD)


# speedup vs baseline: 2.3032x; 1.0052x over previous
import functools

import jax
import jax.numpy as jnp
from jax import lax
from jax.experimental import pallas as pl
from jax.experimental.pallas import tpu as pltpu
from jax.experimental.pallas import tpu_sc as plsc

V = 100001
T_TBL = 1001
CV = 257
WD = 300
CD = 200
K = 5
HD = WD + CD
WORD_RANGE = V - T_TBL
B, L, C = 64, 200, 16
NTOK = B * L

_NC, _NS = 2, 16
_NW = _NC * _NS
_BPW = NTOK // _NW
_CHUNK = 80
_NCH = _BPW // _CHUNK


WDP = 384

_PADBLK = 1024
_PADGRID = -(-V // _PADBLK)


def _pad_body(src_ref, dst_ref):
    dst_ref[:, :WD] = src_ref[...]
    dst_ref[:, WD:] = jnp.zeros((_PADBLK, WDP - WD), jnp.float32)


def _pad_table(word_table):
    return pl.pallas_call(
        _pad_body,
        grid=(_PADGRID,),
        in_specs=[pl.BlockSpec((_PADBLK, WD), lambda i: (i, 0))],
        out_specs=pl.BlockSpec((_PADBLK, WDP), lambda i: (i, 0)),
        out_shape=jax.ShapeDtypeStruct((V, WDP), jnp.float32),
    )(word_table)


def _sc_gather_body(table_hbm, idx_hbm, out_hbm, idx_v, rows_v, sem):
    wid = lax.axis_index("s") * _NC + lax.axis_index("c")
    base = wid * _BPW
    for j in range(_NCH):
        pltpu.sync_copy(idx_hbm.at[pl.ds(base + j * _CHUNK, _CHUNK)], idx_v.at[j])
    for j in range(_NCH):
        pltpu.async_copy(
            table_hbm.at[idx_v.at[j]],
            rows_v.at[j % 2],
            sem,
        ).wait()
        pltpu.sync_copy(rows_v.at[j % 2], out_hbm.at[pl.ds(base + j * _CHUNK, _CHUNK)])


def _sc_word_gather(word_table_padded, flat_words):
    mesh = plsc.VectorSubcoreMesh(core_axis_name="c", subcore_axis_name="s")
    f = pl.kernel(
        _sc_gather_body,
        out_type=jax.ShapeDtypeStruct((NTOK, WDP), jnp.float32),
        mesh=mesh,
        scratch_types=[
            pltpu.VMEM((_NCH, _CHUNK), jnp.int32),
            pltpu.VMEM((2, _CHUNK, WDP), jnp.float32),
            pltpu.SemaphoreType.DMA,
        ],
    )
    return f(word_table_padded, flat_words)


TBLK = 256
GRID = NTOK // TBLK


def _tc_body(we_ref, w_ref, ch_ref, tt_ref, ct_ref, cw_ref, cb_ref,
             w1t_ref, b1t_ref, w1h_ref, b1h_ref,
             w2t_ref, b2t_ref, w2h_ref, b2h_ref,
             out_ref, cep_ref):
    f32 = jnp.float32
    bf16 = jnp.bfloat16
    words = w_ref[0, 0, :]
    tidx = jnp.maximum(words - WORD_RANGE, 0)
    oh_t = (lax.broadcasted_iota(jnp.int32, (TBLK, T_TBL), 1)
            == tidx[:, None]).astype(bf16)
    te = jnp.dot(oh_t, tt_ref[...], preferred_element_type=f32)
    x_we = we_ref[...][:, :WD] + jnp.maximum(te, 0.0)

    ch = ch_ref[0]
    oh_c = (lax.broadcasted_iota(jnp.int32, (TBLK, C, CV), 2)
            == ch[:, :, None]).astype(bf16)
    ce = jnp.dot(oh_c.reshape(TBLK * C, CV), ct_ref[...],
                 preferred_element_type=f32)

    cep_ref[:, 0:2, :] = jnp.zeros((TBLK, 2, CD), bf16)
    cep_ref[:, 2:2 + C, :] = ce.astype(bf16).reshape(TBLK, C, CD)
    cep_ref[:, 2 + C:, :] = jnp.zeros((TBLK, 2, CD), bf16)

    acc = jnp.zeros((TBLK * C, CD), f32)
    for k in range(K):
        sl = cep_ref[:, k:k + C, :].reshape(TBLK * C, CD)
        acc = acc + jnp.dot(sl, cw_ref[k], preferred_element_type=f32)
    acc = jnp.maximum(acc + cb_ref[0], 0.0)
    cpool = jnp.max(acc.reshape(TBLK, C, CD), axis=1)

    x = jnp.concatenate([x_we, cpool], axis=1)
    for wt_ref, bt_ref, wh_ref, bh_ref in (
            (w1t_ref, b1t_ref, w1h_ref, b1h_ref),
            (w2t_ref, b2t_ref, w2h_ref, b2h_ref)):
        xb = x.astype(bf16)
        t = jax.nn.sigmoid(jnp.dot(xb, wt_ref[...], preferred_element_type=f32)
                           + bt_ref[0])
        h = jnp.maximum(jnp.dot(xb, wh_ref[...], preferred_element_type=f32)
                        + bh_ref[0], 0.0)
        x = h * t + x * (1.0 - t)
    out_ref[...] = x


def _full(shape):
    nd = len(shape)
    return pl.BlockSpec(shape, lambda i, _n=nd: (0,) * _n)


def _tc_fused(we, flat_words, chars2d, trainable_table, char_table, conv_w,
              conv_b, hw1_wt, hw1_bt, hw1_wh, hw1_bh, hw2_wt, hw2_bt,
              hw2_wh, hw2_bh, interpret=False):
    words_r = flat_words.reshape(GRID, 1, TBLK)
    chars_r = chars2d.reshape(GRID, TBLK, C)
    return pl.pallas_call(
        _tc_body,
        grid=(GRID,),
        in_specs=[
            pl.BlockSpec((TBLK, WDP), lambda i: (i, 0)),
            pl.BlockSpec((1, 1, TBLK), lambda i: (i, 0, 0)),
            pl.BlockSpec((1, TBLK, C), lambda i: (i, 0, 0)),
            _full((T_TBL, WD)),
            _full((CV, CD)),
            _full((K, CD, CD)),
            _full((1, CD)),
            _full((HD, HD)), _full((1, HD)), _full((HD, HD)), _full((1, HD)),
            _full((HD, HD)), _full((1, HD)), _full((HD, HD)), _full((1, HD)),
        ],
        out_specs=pl.BlockSpec((TBLK, HD), lambda i: (i, 0)),
        out_shape=jax.ShapeDtypeStruct((NTOK, HD), jnp.float32),
        scratch_shapes=[pltpu.VMEM((TBLK, C + 4, CD), jnp.bfloat16)],
        interpret=interpret,
    )(we, words_r, chars_r,
      trainable_table.astype(jnp.bfloat16), char_table.astype(jnp.bfloat16),
      conv_w.astype(jnp.bfloat16), conv_b.reshape(1, CD),
      hw1_wt.astype(jnp.bfloat16), hw1_bt.reshape(1, HD),
      hw1_wh.astype(jnp.bfloat16), hw1_bh.reshape(1, HD),
      hw2_wt.astype(jnp.bfloat16), hw2_bt.reshape(1, HD),
      hw2_wh.astype(jnp.bfloat16), hw2_bh.reshape(1, HD))


def kernel(words, chars, word_table, trainable_table, char_table, conv_w,
           conv_b, hw1_wt, hw1_bt, hw1_wh, hw1_bh, hw2_wt, hw2_bt, hw2_wh,
           hw2_bh):
    flat_words = words.reshape(NTOK)
    chars2d = chars.reshape(NTOK, C)
    wt_pad = _pad_table(word_table)
    we = _sc_word_gather(wt_pad, flat_words)
    out = _tc_fused(we, flat_words, chars2d, trainable_table, char_table,
                    conv_w, conv_b, hw1_wt, hw1_bt, hw1_wh, hw1_bh,
                    hw2_wt, hw2_bt, hw2_wh, hw2_bh)
    return out.reshape(B, L, HD)


# speedup vs baseline: 2.3940x; 1.0394x over previous
import functools

import jax
import jax.numpy as jnp
from jax import lax
from jax.experimental import pallas as pl
from jax.experimental.pallas import tpu as pltpu
from jax.experimental.pallas import tpu_sc as plsc

V = 100001
T_TBL = 1001
CV = 257
WD = 300
CD = 200
K = 5
HD = WD + CD
WORD_RANGE = V - T_TBL
B, L, C = 64, 200, 16
NTOK = B * L

_NC, _NS = 2, 16
_NW = _NC * _NS
_BPW = NTOK // _NW
_CHUNK = 80
_NCH = _BPW // _CHUNK


WDP = 384

_PADBLK = 1024
_PADGRID = -(-V // _PADBLK)


def _pad_body(src_ref, dst_ref):
    dst_ref[:, :WD] = src_ref[...]
    dst_ref[:, WD:] = jnp.zeros((_PADBLK, WDP - WD), jnp.float32)


def _pad_table(word_table):
    return pl.pallas_call(
        _pad_body,
        grid=(_PADGRID,),
        in_specs=[pl.BlockSpec((_PADBLK, WD), lambda i: (i, 0))],
        out_specs=pl.BlockSpec((_PADBLK, WDP), lambda i: (i, 0)),
        out_shape=jax.ShapeDtypeStruct((V, WDP), jnp.float32),
    )(word_table)


def _sc_gather_body(table_hbm, idx_hbm, out_hbm, idx_v, rows_v, sem):
    wid = lax.axis_index("s") * _NC + lax.axis_index("c")
    base = wid * _BPW
    for j in range(_NCH):
        pltpu.sync_copy(idx_hbm.at[pl.ds(base + j * _CHUNK, _CHUNK)], idx_v.at[j])
    for j in range(_NCH):
        pltpu.async_copy(
            table_hbm.at[idx_v.at[j]],
            rows_v.at[j % 2],
            sem,
        ).wait()
        pltpu.sync_copy(rows_v.at[j % 2], out_hbm.at[pl.ds(base + j * _CHUNK, _CHUNK)])


def _sc_word_gather(word_table_padded, flat_words):
    mesh = plsc.VectorSubcoreMesh(core_axis_name="c", subcore_axis_name="s")
    f = pl.kernel(
        _sc_gather_body,
        out_type=jax.ShapeDtypeStruct((NTOK, WDP), jnp.float32),
        mesh=mesh,
        scratch_types=[
            pltpu.VMEM((_NCH, _CHUNK), jnp.int32),
            pltpu.VMEM((2, _CHUNK, WDP), jnp.float32),
            pltpu.SemaphoreType.DMA,
        ],
    )
    return f(word_table_padded, flat_words)


TBLK = 256
GRID = NTOK // TBLK


def _tc_body(we_ref, w_ref, ch_ref, tt_ref, ct_ref, cw_ref, cb_ref,
             w1t_ref, b1t_ref, w1h_ref, b1h_ref,
             w2t_ref, b2t_ref, w2h_ref, b2h_ref,
             out_ref, cep_ref):
    f32 = jnp.float32
    bf16 = jnp.bfloat16
    words = w_ref[0, 0, :]
    tidx = jnp.maximum(words - WORD_RANGE, 0)
    oh_t = (lax.broadcasted_iota(jnp.int32, (TBLK, T_TBL), 1)
            == tidx[:, None]).astype(bf16)
    te = jnp.dot(oh_t, tt_ref[...], preferred_element_type=f32)
    x_we = we_ref[...][:, :WD] + jnp.maximum(te, 0.0)

    ch_t = ch_ref[0].T
    oh_c = (lax.broadcasted_iota(jnp.int32, (C, TBLK, CV), 2)
            == ch_t[:, :, None]).astype(bf16)
    ce = jnp.dot(oh_c.reshape(C * TBLK, CV), ct_ref[...],
                 preferred_element_type=f32)

    cep_ref[0:2] = jnp.zeros((2, TBLK, CD), bf16)
    cep_ref[2:2 + C] = ce.astype(bf16).reshape(C, TBLK, CD)
    cep_ref[2 + C:] = jnp.zeros((2, TBLK, CD), bf16)

    acc = jnp.zeros((C * TBLK, CD), f32)
    for k in range(K):
        sl = cep_ref[k:k + C].reshape(C * TBLK, CD)
        acc = acc + jnp.dot(sl, cw_ref[k], preferred_element_type=f32)
    acc = jnp.maximum(acc + cb_ref[0], 0.0)
    cpool = jnp.max(acc.reshape(C, TBLK, CD), axis=0)

    x = jnp.concatenate([x_we, cpool], axis=1)
    for wt_ref, bt_ref, wh_ref, bh_ref in (
            (w1t_ref, b1t_ref, w1h_ref, b1h_ref),
            (w2t_ref, b2t_ref, w2h_ref, b2h_ref)):
        xb = x.astype(bf16)
        t = jax.nn.sigmoid(jnp.dot(xb, wt_ref[...], preferred_element_type=f32)
                           + bt_ref[0])
        h = jnp.maximum(jnp.dot(xb, wh_ref[...], preferred_element_type=f32)
                        + bh_ref[0], 0.0)
        x = h * t + x * (1.0 - t)
    out_ref[...] = x


def _full(shape):
    nd = len(shape)
    return pl.BlockSpec(shape, lambda i, _n=nd: (0,) * _n)


def _tc_fused(we, flat_words, chars2d, trainable_table, char_table, conv_w,
              conv_b, hw1_wt, hw1_bt, hw1_wh, hw1_bh, hw2_wt, hw2_bt,
              hw2_wh, hw2_bh, interpret=False):
    words_r = flat_words.reshape(GRID, 1, TBLK)
    chars_r = chars2d.reshape(GRID, TBLK, C)
    return pl.pallas_call(
        _tc_body,
        grid=(GRID,),
        in_specs=[
            pl.BlockSpec((TBLK, WDP), lambda i: (i, 0)),
            pl.BlockSpec((1, 1, TBLK), lambda i: (i, 0, 0)),
            pl.BlockSpec((1, TBLK, C), lambda i: (i, 0, 0)),
            _full((T_TBL, WD)),
            _full((CV, CD)),
            _full((K, CD, CD)),
            _full((1, CD)),
            _full((HD, HD)), _full((1, HD)), _full((HD, HD)), _full((1, HD)),
            _full((HD, HD)), _full((1, HD)), _full((HD, HD)), _full((1, HD)),
        ],
        out_specs=pl.BlockSpec((TBLK, HD), lambda i: (i, 0)),
        out_shape=jax.ShapeDtypeStruct((NTOK, HD), jnp.float32),
        scratch_shapes=[pltpu.VMEM((C + 4, TBLK, CD), jnp.bfloat16)],
        interpret=interpret,
    )(we, words_r, chars_r,
      trainable_table.astype(jnp.bfloat16), char_table.astype(jnp.bfloat16),
      conv_w.astype(jnp.bfloat16), conv_b.reshape(1, CD),
      hw1_wt.astype(jnp.bfloat16), hw1_bt.reshape(1, HD),
      hw1_wh.astype(jnp.bfloat16), hw1_bh.reshape(1, HD),
      hw2_wt.astype(jnp.bfloat16), hw2_bt.reshape(1, HD),
      hw2_wh.astype(jnp.bfloat16), hw2_bh.reshape(1, HD))


def kernel(words, chars, word_table, trainable_table, char_table, conv_w,
           conv_b, hw1_wt, hw1_bt, hw1_wh, hw1_bh, hw2_wt, hw2_bt, hw2_wh,
           hw2_bh):
    flat_words = words.reshape(NTOK)
    chars2d = chars.reshape(NTOK, C)
    wt_pad = _pad_table(word_table)
    we = _sc_word_gather(wt_pad, flat_words)
    out = _tc_fused(we, flat_words, chars2d, trainable_table, char_table,
                    conv_w, conv_b, hw1_wt, hw1_bt, hw1_wh, hw1_bh,
                    hw2_wt, hw2_bt, hw2_wh, hw2_bh)
    return out.reshape(B, L, HD)


# speedup vs baseline: 2.7837x; 1.1628x over previous
import functools

import jax
import jax.numpy as jnp
from jax import lax
from jax.experimental import pallas as pl
from jax.experimental.pallas import tpu as pltpu
from jax.experimental.pallas import tpu_sc as plsc

V = 100001
T_TBL = 1001
CV = 257
WD = 300
CD = 200
K = 5
HD = WD + CD
WORD_RANGE = V - T_TBL
B, L, C = 64, 200, 16
NTOK = B * L

_NC, _NS = 2, 16
_NW = _NC * _NS
_BPW = NTOK // _NW
_CHUNK = 80
_NCH = _BPW // _CHUNK


WDP = 384

_PADBLK = 2048
_PADGRID = -(-V // _PADBLK)


def _pad_body(src_ref, dst_ref):
    dst_ref[:, :WD] = src_ref[...]
    dst_ref[:, WD:] = jnp.zeros((_PADBLK, WDP - WD), jnp.float32)


def _pad_table(word_table):
    return pl.pallas_call(
        _pad_body,
        grid=(_PADGRID,),
        in_specs=[pl.BlockSpec((_PADBLK, WD), lambda i: (i, 0))],
        out_specs=pl.BlockSpec((_PADBLK, WDP), lambda i: (i, 0)),
        out_shape=jax.ShapeDtypeStruct((V, WDP), jnp.float32),
    )(word_table)


def _sc_gather_body(table_hbm, idx_hbm, out_hbm, idx_v, rows_v, sem):
    wid = lax.axis_index("s") * _NC + lax.axis_index("c")
    base = wid * _BPW
    for j in range(_NCH):
        pltpu.sync_copy(idx_hbm.at[pl.ds(base + j * _CHUNK, _CHUNK)], idx_v.at[j])
    for j in range(_NCH):
        pltpu.async_copy(
            table_hbm.at[idx_v.at[j]],
            rows_v.at[j % 2],
            sem,
        ).wait()
        pltpu.sync_copy(rows_v.at[j % 2], out_hbm.at[pl.ds(base + j * _CHUNK, _CHUNK)])


def _sc_word_gather(word_table_padded, flat_words):
    mesh = plsc.VectorSubcoreMesh(core_axis_name="c", subcore_axis_name="s")
    f = pl.kernel(
        _sc_gather_body,
        out_type=jax.ShapeDtypeStruct((NTOK, WDP), jnp.float32),
        mesh=mesh,
        scratch_types=[
            pltpu.VMEM((_NCH, _CHUNK), jnp.int32),
            pltpu.VMEM((2, _CHUNK, WDP), jnp.float32),
            pltpu.SemaphoreType.DMA,
        ],
    )
    return f(word_table_padded, flat_words)


TBLK = 512
GRID = NTOK // TBLK


def _tc_body(we_ref, w_ref, ch_ref, tt_ref, ct_ref, cw_ref, cb_ref,
             w1t_ref, b1t_ref, w1h_ref, b1h_ref,
             w2t_ref, b2t_ref, w2h_ref, b2h_ref,
             out_ref, cep_ref):
    f32 = jnp.float32
    bf16 = jnp.bfloat16
    words = w_ref[0, 0, :]
    tidx = jnp.maximum(words - WORD_RANGE, 0)
    oh_t = (lax.broadcasted_iota(jnp.int32, (TBLK, T_TBL), 1)
            == tidx[:, None]).astype(bf16)
    te = jnp.dot(oh_t, tt_ref[...], preferred_element_type=f32)
    x_we = we_ref[...][:, :WD] + jnp.maximum(te, 0.0)

    ch_t = ch_ref[0].T
    oh_c = (lax.broadcasted_iota(jnp.int32, (C, TBLK, CV - 1), 2)
            == ch_t[:, :, None]).astype(bf16)
    ce = jnp.dot(oh_c.reshape(C * TBLK, CV - 1), ct_ref[:CV - 1, :],
                 preferred_element_type=f32)
    m256 = (ch_t == CV - 1).astype(f32)[:, :, None]
    ce3 = ce.reshape(C, TBLK, CD) + m256 * ct_ref[CV - 1:CV, :].astype(f32)

    cep_ref[0:2] = jnp.zeros((2, TBLK, CD), bf16)
    cep_ref[2:2 + C] = ce3.astype(bf16)
    cep_ref[2 + C:] = jnp.zeros((2, TBLK, CD), bf16)

    acc = jnp.zeros((C * TBLK, CD), f32)
    for k in range(K):
        sl = cep_ref[k:k + C].reshape(C * TBLK, CD)
        acc = acc + jnp.dot(sl, cw_ref[k], preferred_element_type=f32)
    acc = jnp.maximum(acc + cb_ref[0], 0.0)
    cpool = jnp.max(acc.reshape(C, TBLK, CD), axis=0)

    x = jnp.concatenate([x_we, cpool], axis=1)
    for wt_ref, bt_ref, wh_ref, bh_ref in (
            (w1t_ref, b1t_ref, w1h_ref, b1h_ref),
            (w2t_ref, b2t_ref, w2h_ref, b2h_ref)):
        xb = x.astype(bf16)
        t = jax.nn.sigmoid(jnp.dot(xb, wt_ref[...], preferred_element_type=f32)
                           + bt_ref[0])
        h = jnp.maximum(jnp.dot(xb, wh_ref[...], preferred_element_type=f32)
                        + bh_ref[0], 0.0)
        x = h * t + x * (1.0 - t)
    out_ref[...] = x


def _full(shape):
    nd = len(shape)
    return pl.BlockSpec(shape, lambda i, _n=nd: (0,) * _n)


def _tc_fused(we, flat_words, chars2d, trainable_table, char_table, conv_w,
              conv_b, hw1_wt, hw1_bt, hw1_wh, hw1_bh, hw2_wt, hw2_bt,
              hw2_wh, hw2_bh, interpret=False):
    words_r = flat_words.reshape(GRID, 1, TBLK)
    chars_r = chars2d.reshape(GRID, TBLK, C)
    return pl.pallas_call(
        _tc_body,
        grid=(GRID,),
        in_specs=[
            pl.BlockSpec((TBLK, WDP), lambda i: (i, 0)),
            pl.BlockSpec((1, 1, TBLK), lambda i: (i, 0, 0)),
            pl.BlockSpec((1, TBLK, C), lambda i: (i, 0, 0)),
            _full((T_TBL, WD)),
            _full((CV, CD)),
            _full((K, CD, CD)),
            _full((1, CD)),
            _full((HD, HD)), _full((1, HD)), _full((HD, HD)), _full((1, HD)),
            _full((HD, HD)), _full((1, HD)), _full((HD, HD)), _full((1, HD)),
        ],
        out_specs=pl.BlockSpec((TBLK, HD), lambda i: (i, 0)),
        out_shape=jax.ShapeDtypeStruct((NTOK, HD), jnp.float32),
        scratch_shapes=[pltpu.VMEM((C + 4, TBLK, CD), jnp.bfloat16)],
        interpret=interpret,
    )(we, words_r, chars_r,
      trainable_table.astype(jnp.bfloat16), char_table.astype(jnp.bfloat16),
      conv_w.astype(jnp.bfloat16), conv_b.reshape(1, CD),
      hw1_wt.astype(jnp.bfloat16), hw1_bt.reshape(1, HD),
      hw1_wh.astype(jnp.bfloat16), hw1_bh.reshape(1, HD),
      hw2_wt.astype(jnp.bfloat16), hw2_bt.reshape(1, HD),
      hw2_wh.astype(jnp.bfloat16), hw2_bh.reshape(1, HD))


def kernel(words, chars, word_table, trainable_table, char_table, conv_w,
           conv_b, hw1_wt, hw1_bt, hw1_wh, hw1_bh, hw2_wt, hw2_bt, hw2_wh,
           hw2_bh):
    flat_words = words.reshape(NTOK)
    chars2d = chars.reshape(NTOK, C)
    wt_pad = _pad_table(word_table)
    we = _sc_word_gather(wt_pad, flat_words)
    out = _tc_fused(we, flat_words, chars2d, trainable_table, char_table,
                    conv_w, conv_b, hw1_wt, hw1_bt, hw1_wh, hw1_bh,
                    hw2_wt, hw2_bt, hw2_wh, hw2_bh)
    return out.reshape(B, L, HD)


# speedup vs baseline: 3.0348x; 1.0902x over previous
import functools

import jax
import jax.numpy as jnp
from jax import lax
from jax.experimental import pallas as pl
from jax.experimental.pallas import tpu as pltpu
from jax.experimental.pallas import tpu_sc as plsc

V = 100001
T_TBL = 1001
CV = 257
WD = 300
CD = 200
K = 5
HD = WD + CD
WORD_RANGE = V - T_TBL
B, L, C = 64, 200, 16
NTOK = B * L

_NC, _NS = 2, 16
_NW = _NC * _NS
_BPW = NTOK // _NW
_CHUNK = 80
_NCH = _BPW // _CHUNK


WDP = 384

_HEADW = 256
_TAILW = WD - _HEADW
_PADBLK = 2048
_PADGRID = -(-V // _PADBLK)


def _pad_body(src_ref, dst_ref):
    dst_ref[:, :_TAILW] = src_ref[:, :_TAILW]
    dst_ref[:, _TAILW:] = jnp.zeros((_PADBLK, 128 - _TAILW), jnp.float32)


def _tail_table(word_table):
    return pl.pallas_call(
        _pad_body,
        grid=(_PADGRID,),
        in_specs=[pl.BlockSpec((_PADBLK, 128), lambda i: (i, 2))],
        out_specs=pl.BlockSpec((_PADBLK, 128), lambda i: (i, 0)),
        out_shape=jax.ShapeDtypeStruct((V, 128), jnp.float32),
    )(word_table)


def _sc_gather_body(table_hbm, tail_hbm, idx_hbm, out_hbm,
                    idx_v, head_v, tail_v, sem, sem2):
    wid = lax.axis_index("s") * _NC + lax.axis_index("c")
    base = wid * _BPW
    for j in range(_NCH):
        pltpu.sync_copy(idx_hbm.at[pl.ds(base + j * _CHUNK, _CHUNK)], idx_v.at[j])
    for j in range(_NCH):
        b = j % 2
        h1 = pltpu.async_copy(
            table_hbm.at[idx_v.at[j], pl.ds(0, _HEADW)], head_v.at[b], sem)
        h2 = pltpu.async_copy(tail_hbm.at[idx_v.at[j]], tail_v.at[b], sem2)
        h1.wait()
        h2.wait()
        row0 = base + j * _CHUNK
        pltpu.sync_copy(head_v.at[b],
                        out_hbm.at[pl.ds(row0, _CHUNK), pl.ds(0, _HEADW)])
        pltpu.sync_copy(tail_v.at[b],
                        out_hbm.at[pl.ds(row0, _CHUNK), pl.ds(_HEADW, 128)])


def _sc_word_gather(word_table, tail_tbl, flat_words):
    mesh = plsc.VectorSubcoreMesh(core_axis_name="c", subcore_axis_name="s")
    f = pl.kernel(
        _sc_gather_body,
        out_type=jax.ShapeDtypeStruct((NTOK, WDP), jnp.float32),
        mesh=mesh,
        scratch_types=[
            pltpu.VMEM((_NCH, _CHUNK), jnp.int32),
            pltpu.VMEM((2, _CHUNK, _HEADW), jnp.float32),
            pltpu.VMEM((2, _CHUNK, 128), jnp.float32),
            pltpu.SemaphoreType.DMA,
            pltpu.SemaphoreType.DMA,
        ],
    )
    return f(word_table, tail_tbl, flat_words)


TBLK = 512
GRID = NTOK // TBLK


def _tc_body(we_ref, w_ref, ch_ref, tt_ref, ct_ref, cw_ref, cb_ref,
             w1t_ref, b1t_ref, w1h_ref, b1h_ref,
             w2t_ref, b2t_ref, w2h_ref, b2h_ref,
             out_ref, cep_ref):
    f32 = jnp.float32
    bf16 = jnp.bfloat16
    words = w_ref[0, 0, :]
    tidx = jnp.maximum(words - WORD_RANGE, 0)
    oh_t = (lax.broadcasted_iota(jnp.int32, (TBLK, T_TBL), 1)
            == tidx[:, None]).astype(bf16)
    te = jnp.dot(oh_t, tt_ref[...], preferred_element_type=f32)
    x_we = we_ref[...][:, :WD] + jnp.maximum(te, 0.0)

    ch_t = ch_ref[0].T
    oh_c = (lax.broadcasted_iota(jnp.int32, (C, TBLK, CV - 1), 2)
            == ch_t[:, :, None]).astype(bf16)
    ce = jnp.dot(oh_c.reshape(C * TBLK, CV - 1), ct_ref[:CV - 1, :],
                 preferred_element_type=f32)
    m256 = (ch_t == CV - 1).astype(f32)[:, :, None]
    ce3 = ce.reshape(C, TBLK, CD) + m256 * ct_ref[CV - 1:CV, :].astype(f32)

    cep_ref[0:2] = jnp.zeros((2, TBLK, CD), bf16)
    cep_ref[2:2 + C] = ce3.astype(bf16)
    cep_ref[2 + C:] = jnp.zeros((2, TBLK, CD), bf16)

    acc = jnp.zeros((C * TBLK, CD), f32)
    for k in range(K):
        sl = cep_ref[k:k + C].reshape(C * TBLK, CD)
        acc = acc + jnp.dot(sl, cw_ref[k], preferred_element_type=f32)
    acc = jnp.maximum(acc + cb_ref[0], 0.0)
    cpool = jnp.max(acc.reshape(C, TBLK, CD), axis=0)

    x = jnp.concatenate([x_we, cpool], axis=1)
    for wt_ref, bt_ref, wh_ref, bh_ref in (
            (w1t_ref, b1t_ref, w1h_ref, b1h_ref),
            (w2t_ref, b2t_ref, w2h_ref, b2h_ref)):
        xb = x.astype(bf16)
        t = jax.nn.sigmoid(jnp.dot(xb, wt_ref[...], preferred_element_type=f32)
                           + bt_ref[0])
        h = jnp.maximum(jnp.dot(xb, wh_ref[...], preferred_element_type=f32)
                        + bh_ref[0], 0.0)
        x = h * t + x * (1.0 - t)
    out_ref[...] = x


def _full(shape):
    nd = len(shape)
    return pl.BlockSpec(shape, lambda i, _n=nd: (0,) * _n)


def _tc_fused(we, flat_words, chars2d, trainable_table, char_table, conv_w,
              conv_b, hw1_wt, hw1_bt, hw1_wh, hw1_bh, hw2_wt, hw2_bt,
              hw2_wh, hw2_bh, interpret=False):
    words_r = flat_words.reshape(GRID, 1, TBLK)
    chars_r = chars2d.reshape(GRID, TBLK, C)
    return pl.pallas_call(
        _tc_body,
        grid=(GRID,),
        in_specs=[
            pl.BlockSpec((TBLK, WDP), lambda i: (i, 0)),
            pl.BlockSpec((1, 1, TBLK), lambda i: (i, 0, 0)),
            pl.BlockSpec((1, TBLK, C), lambda i: (i, 0, 0)),
            _full((T_TBL, WD)),
            _full((CV, CD)),
            _full((K, CD, CD)),
            _full((1, CD)),
            _full((HD, HD)), _full((1, HD)), _full((HD, HD)), _full((1, HD)),
            _full((HD, HD)), _full((1, HD)), _full((HD, HD)), _full((1, HD)),
        ],
        out_specs=pl.BlockSpec((TBLK, HD), lambda i: (i, 0)),
        out_shape=jax.ShapeDtypeStruct((NTOK, HD), jnp.float32),
        scratch_shapes=[pltpu.VMEM((C + 4, TBLK, CD), jnp.bfloat16)],
        interpret=interpret,
    )(we, words_r, chars_r,
      trainable_table.astype(jnp.bfloat16), char_table.astype(jnp.bfloat16),
      conv_w.astype(jnp.bfloat16), conv_b.reshape(1, CD),
      hw1_wt.astype(jnp.bfloat16), hw1_bt.reshape(1, HD),
      hw1_wh.astype(jnp.bfloat16), hw1_bh.reshape(1, HD),
      hw2_wt.astype(jnp.bfloat16), hw2_bt.reshape(1, HD),
      hw2_wh.astype(jnp.bfloat16), hw2_bh.reshape(1, HD))


def kernel(words, chars, word_table, trainable_table, char_table, conv_w,
           conv_b, hw1_wt, hw1_bt, hw1_wh, hw1_bh, hw2_wt, hw2_bt, hw2_wh,
           hw2_bh):
    flat_words = words.reshape(NTOK)
    chars2d = chars.reshape(NTOK, C)
    tail_tbl = _tail_table(word_table)
    we = _sc_word_gather(word_table, tail_tbl, flat_words)
    out = _tc_fused(we, flat_words, chars2d, trainable_table, char_table,
                    conv_w, conv_b, hw1_wt, hw1_bt, hw1_wh, hw1_bh,
                    hw2_wt, hw2_bt, hw2_wh, hw2_bh)
    return out.reshape(B, L, HD)
